# baseline (speedup 1.0000x reference)
import functools
import math

import numpy as np
import jax
import jax.numpy as jnp
from jax import lax
from jax.experimental import pallas as pl
from jax.experimental.pallas import tpu as pltpu
from jax.experimental.pallas import tpu_sc as plsc

VOCAB = 1000000
D = 64
H = 32
P = 128
B = 1024
S = 200
NTOK = B * S
QTOK = NTOK // 4

VB = 16384
VB4 = VB // 4
NFULL = VOCAB // VB
TAIL_V0 = NFULL * VB
TAIL = VOCAB - TAIL_V0
TAIL4 = TAIL // 4
NPREP = NFULL + 1


def _rne16(u):
    return (u + 0x7FFF + ((u >> 16) & 1)) >> 16


def _tc_prep(table_t):
    def body(t_ref, o_ref):
        et = jnp.transpose(t_ref[...], (1, 0))
        ua = lax.bitcast_convert_type(et[:, :H], jnp.uint32)
        ub = lax.bitcast_convert_type(et[:, H:], jnp.uint32)
        word = (_rne16(ua) << 16) | _rne16(ub)
        wf32 = lax.bitcast_convert_type(word, jnp.float32)
        i = pl.program_id(0)

        @pl.when(i < NFULL)
        def _():
            for k in range(4):
                o_ref[:, k * H:(k + 1) * H] = wf32[k * VB4:(k + 1) * VB4, :]

        @pl.when(i == NFULL)
        def _():
            for k in range(4):
                o_ref[:TAIL4, k * H:(k + 1) * H] = (
                    wf32[k * TAIL4:(k + 1) * TAIL4, :])

    return pl.pallas_call(
        body,
        grid=(NPREP,),
        in_specs=[pl.BlockSpec((D, VB), lambda i: (0, i))],
        out_specs=pl.BlockSpec((VB4, P), lambda i: (i, 0)),
        out_shape=jax.ShapeDtypeStruct((VOCAB // 4, P), jnp.float32),
    )(table_t)


NC, NS = 2, 16
NW = NC * NS
TOK_PER_W = NTOK // NW
W_PER_Q = NW // 4
CHUNK = 1280
NCHUNK = TOK_PER_W // CHUNK


def _sc_gather(table_packed, idx_flat):
    mesh = plsc.VectorSubcoreMesh(core_axis_name="c", subcore_axis_name="s")

    @functools.partial(
        pl.kernel,
        mesh=mesh,
        out_type=jax.ShapeDtypeStruct((QTOK, P), jnp.float32),
        scratch_types=[
            pltpu.VMEM((CHUNK,), jnp.int32),
            pltpu.VMEM((CHUNK, H), jnp.float32),
            pltpu.SemaphoreType.DMA,
        ],
        compiler_params=pltpu.CompilerParams(use_tc_tiling_on_sc=False),
    )
    def k(table_hbm, idx_hbm, out_hbm, idx_v, rows_v, sem):
        wid = lax.axis_index("s") * NC + lax.axis_index("c")
        base = wid * TOK_PER_W
        row_base = (wid % W_PER_Q) * TOK_PER_W

        def body(i, carry):
            off = base + i * CHUNK
            row = row_base + i * CHUNK
            pltpu.sync_copy(idx_hbm.at[pl.ds(off, CHUNK)], idx_v)
            pltpu.async_copy(table_hbm.at[idx_v], rows_v, sem).wait()
            for q in range(4):
                @pl.when(wid // W_PER_Q == q)
                def _():
                    pltpu.sync_copy(
                        rows_v,
                        out_hbm.at[pl.ds(row, CHUNK), pl.ds(q * H, H)])
            return carry

        lax.fori_loop(0, NCHUNK, body, 0)

    return k(table_packed, idx_flat)


TB = 3200
NSTEPS = QTOK // TB


def _positional_encoding_np(seq_len, embed_dim):
    position = np.arange(0, seq_len, dtype=np.float32)[:, None]
    div_term = np.exp(
        np.arange(0, embed_dim, 2, dtype=np.float32)
        * (-math.log(10000.0) / embed_dim))
    pe = np.zeros((seq_len, embed_dim), dtype=np.float32)
    pe[:, 0::2] = np.sin(position * div_term)
    pe[:, 1::2] = np.cos(position * div_term)
    return pe


_PE_FULL = np.tile(_positional_encoding_np(S, D), (TB // S, 1))
_PE_A = np.ascontiguousarray(_PE_FULL[:, :H])
_PE_B = np.ascontiguousarray(_PE_FULL[:, H:])


def _tc_ln_proj(emb4, pe_a, pe_b, wfa, wfb, bias):
    inv_d = 1.0 / D
    hi = np.uint32(0xFFFF0000)

    def body(e_ref, pa_ref, pb_ref, wa_ref, wb_ref, b_ref, o_ref):
        pa = pa_ref[...]
        pb = pb_ref[...]
        wa = wa_ref[...]
        wb = wb_ref[...]
        bias_v = b_ref[...]
        for q in range(4):
            wq = lax.bitcast_convert_type(
                e_ref[:, q * H:(q + 1) * H], jnp.uint32)
            a = lax.bitcast_convert_type(wq & hi, jnp.float32) + pa
            bb = lax.bitcast_convert_type(wq << 16, jnp.float32) + pb
            mu = (jnp.sum(a, -1, keepdims=True)
                  + jnp.sum(bb, -1, keepdims=True)) * inv_d
            m2 = (jnp.sum(a * a, -1, keepdims=True)
                  + jnp.sum(bb * bb, -1, keepdims=True)) * inv_d
            r = lax.rsqrt(m2 - mu * mu + 1e-5)
            na = (a - mu) * r
            nb = (bb - mu) * r
            o_ref[q] = (
                jnp.dot(na, wa, preferred_element_type=jnp.float32)
                + jnp.dot(nb, wb, preferred_element_type=jnp.float32)
                + bias_v)

    return pl.pallas_call(
        body,
        grid=(NSTEPS,),
        in_specs=[
            pl.BlockSpec((TB, P), lambda i: (i, 0)),
            pl.BlockSpec((TB, H), lambda i: (0, 0)),
            pl.BlockSpec((TB, H), lambda i: (0, 0)),
            pl.BlockSpec((H, P), lambda i: (0, 0)),
            pl.BlockSpec((H, P), lambda i: (0, 0)),
            pl.BlockSpec((1, P), lambda i: (0, 0)),
        ],
        out_specs=pl.BlockSpec((4, TB, P), lambda i: (0, i, 0)),
        out_shape=jax.ShapeDtypeStruct((4, QTOK, P), jnp.float32),
    )(emb4, pe_a, pe_b, wfa, wfb, bias)


def kernel(x, table, gamma, beta, W, b):
    packed = _tc_prep(jnp.transpose(table))
    table_rows = packed.reshape(VOCAB, H)
    v = x.reshape(NTOK).astype(jnp.int32)
    dv = v % VB
    main = (v - dv) + 4 * (dv % VB4) + dv // VB4
    dt = v - TAIL_V0
    tail = TAIL_V0 + 4 * (dt % TAIL4) + dt // TAIL4
    idx_flat = jnp.where(v < TAIL_V0, main, tail)
    emb4 = _sc_gather(table_rows, idx_flat)
    wf = gamma[:, None] * W
    wfa, wfb = wf[:H], wf[H:]
    bias = (beta @ W + b).reshape(1, P)
    out = _tc_ln_proj(emb4, jnp.asarray(_PE_A), jnp.asarray(_PE_B),
                      wfa, wfb, bias)
    return out.reshape(B, S, P)


# speedup vs baseline: 2.0841x; 2.0841x over previous
import functools
import math

import numpy as np
import jax
import jax.numpy as jnp
from jax import lax
from jax.experimental import pallas as pl
from jax.experimental.pallas import tpu as pltpu
from jax.experimental.pallas import tpu_sc as plsc

VOCAB = 1000000
D = 64
H = 32
P = 128
B = 1024
S = 200
NTOK = B * S
QTOK = NTOK // 4

VB = 16384
VB4 = VB // 4
NFULL = VOCAB // VB
TAIL_V0 = NFULL * VB
TAIL = VOCAB - TAIL_V0
TAIL4 = TAIL // 4
NPREP = NFULL + 1


def _rne16(u):
    return (u + 0x7FFF + ((u >> 16) & 1)) >> 16


def _tc_prep(table_t):
    def body(t_ref, o_ref):
        ua = lax.bitcast_convert_type(t_ref[:H, :], jnp.uint32)
        ub = lax.bitcast_convert_type(t_ref[H:, :], jnp.uint32)
        word_t = (_rne16(ua) << 16) | _rne16(ub)
        word = jnp.transpose(word_t, (1, 0))
        wf32 = lax.bitcast_convert_type(word, jnp.float32)
        i = pl.program_id(0)

        @pl.when(i < NFULL)
        def _():
            for k in range(4):
                o_ref[:, k * H:(k + 1) * H] = wf32[k * VB4:(k + 1) * VB4, :]

        @pl.when(i == NFULL)
        def _():
            for k in range(4):
                o_ref[:TAIL4, k * H:(k + 1) * H] = (
                    wf32[k * TAIL4:(k + 1) * TAIL4, :])

    return pl.pallas_call(
        body,
        grid=(NPREP,),
        in_specs=[pl.BlockSpec((D, VB), lambda i: (0, i))],
        out_specs=pl.BlockSpec((VB4, P), lambda i: (i, 0)),
        out_shape=jax.ShapeDtypeStruct((VOCAB // 4, P), jnp.float32),
    )(table_t)


NC, NS = 2, 16
NW = NC * NS
TOK_PER_W = NTOK // NW
W_PER_Q = NW // 4
CHUNK = 1280
NCHUNK = TOK_PER_W // CHUNK


def _sc_gather(table_packed, idx_flat):
    mesh = plsc.VectorSubcoreMesh(core_axis_name="c", subcore_axis_name="s")

    @functools.partial(
        pl.kernel,
        mesh=mesh,
        out_type=jax.ShapeDtypeStruct((QTOK, P), jnp.float32),
        scratch_types=[
            pltpu.VMEM((CHUNK,), jnp.int32),
            pltpu.VMEM((CHUNK, H), jnp.float32),
            pltpu.SemaphoreType.DMA,
        ],
        compiler_params=pltpu.CompilerParams(use_tc_tiling_on_sc=False),
    )
    def k(table_hbm, idx_hbm, out_hbm, idx_v, rows_v, sem):
        wid = lax.axis_index("s") * NC + lax.axis_index("c")
        base = wid * TOK_PER_W
        row_base = (wid % W_PER_Q) * TOK_PER_W

        def body(i, carry):
            off = base + i * CHUNK
            row = row_base + i * CHUNK
            pltpu.sync_copy(idx_hbm.at[pl.ds(off, CHUNK)], idx_v)
            pltpu.async_copy(table_hbm.at[idx_v], rows_v, sem).wait()
            for q in range(4):
                @pl.when(wid // W_PER_Q == q)
                def _():
                    pltpu.sync_copy(
                        rows_v,
                        out_hbm.at[pl.ds(row, CHUNK), pl.ds(q * H, H)])
            return carry

        lax.fori_loop(0, NCHUNK, body, 0)

    return k(table_packed, idx_flat)


TB = 3200
NSTEPS = QTOK // TB


def _positional_encoding_np(seq_len, embed_dim):
    position = np.arange(0, seq_len, dtype=np.float32)[:, None]
    div_term = np.exp(
        np.arange(0, embed_dim, 2, dtype=np.float32)
        * (-math.log(10000.0) / embed_dim))
    pe = np.zeros((seq_len, embed_dim), dtype=np.float32)
    pe[:, 0::2] = np.sin(position * div_term)
    pe[:, 1::2] = np.cos(position * div_term)
    return pe


_PE_FULL = np.tile(_positional_encoding_np(S, D), (TB // S, 1))
_PE_A = np.ascontiguousarray(_PE_FULL[:, :H])
_PE_B = np.ascontiguousarray(_PE_FULL[:, H:])


def _tc_ln_proj(emb4, pe_a4, pe_b4, q128, wfa, wfb, bias):
    inv_d = 1.0 / D
    hi = np.uint32(0xFFFF0000)

    def body(e_ref, pa_ref, pb_ref, q_ref, wa_ref, wb_ref, b_ref, o_ref):
        w = lax.bitcast_convert_type(e_ref[...], jnp.uint32)
        a = lax.bitcast_convert_type(w & hi, jnp.float32) + pa_ref[...]
        bb = lax.bitcast_convert_type(w << 16, jnp.float32) + pb_ref[...]
        q = q_ref[...]
        mu = jnp.dot(a + bb, q, preferred_element_type=jnp.float32) * inv_d
        m2 = jnp.dot(a * a + bb * bb, q,
                     preferred_element_type=jnp.float32) * inv_d
        r = lax.rsqrt(m2 - mu * mu + 1e-5)
        na = (a - mu) * r
        nb = (bb - mu) * r
        wa = wa_ref[...]
        wb = wb_ref[...]
        bias_v = b_ref[...]
        for k in range(4):
            o_ref[k] = (
                jnp.dot(na[:, k * H:(k + 1) * H], wa,
                        preferred_element_type=jnp.float32)
                + jnp.dot(nb[:, k * H:(k + 1) * H], wb,
                          preferred_element_type=jnp.float32)
                + bias_v)

    return pl.pallas_call(
        body,
        grid=(NSTEPS,),
        in_specs=[
            pl.BlockSpec((TB, P), lambda i: (i, 0)),
            pl.BlockSpec((TB, P), lambda i: (0, 0)),
            pl.BlockSpec((TB, P), lambda i: (0, 0)),
            pl.BlockSpec((P, P), lambda i: (0, 0)),
            pl.BlockSpec((H, P), lambda i: (0, 0)),
            pl.BlockSpec((H, P), lambda i: (0, 0)),
            pl.BlockSpec((1, P), lambda i: (0, 0)),
        ],
        out_specs=pl.BlockSpec((4, TB, P), lambda i: (0, i, 0)),
        out_shape=jax.ShapeDtypeStruct((4, QTOK, P), jnp.float32),
    )(emb4, pe_a4, pe_b4, q128, wfa, wfb, bias)


def kernel(x, table, gamma, beta, W, b):
    packed = _tc_prep(jnp.transpose(table))
    table_rows = packed.reshape(VOCAB, H)
    v = x.reshape(NTOK).astype(jnp.int32)
    dv = v % VB
    main = (v - dv) + 4 * (dv % VB4) + dv // VB4
    dt = v - TAIL_V0
    tail = TAIL_V0 + 4 * (dt % TAIL4) + dt // TAIL4
    idx_flat = jnp.where(v < TAIL_V0, main, tail)
    emb4 = _sc_gather(table_rows, idx_flat)
    wf = gamma[:, None] * W
    wfa, wfb = wf[:H], wf[H:]
    bias = (beta @ W + b).reshape(1, P)
    pe_a4 = jnp.asarray(np.tile(_PE_A, (1, 4)))
    pe_b4 = jnp.asarray(np.tile(_PE_B, (1, 4)))
    q128 = jnp.asarray(np.kron(np.eye(4, dtype=np.float32),
                               np.ones((H, H), np.float32)))
    out = _tc_ln_proj(emb4, pe_a4, pe_b4, q128, wfa, wfb, bias)
    return out.reshape(B, S, P)


# speedup vs baseline: 2.0894x; 1.0025x over previous
import functools
import math

import numpy as np
import jax
import jax.numpy as jnp
from jax import lax
from jax.experimental import pallas as pl
from jax.experimental.pallas import tpu as pltpu
from jax.experimental.pallas import tpu_sc as plsc

VOCAB = 1000000
D = 64
H = 32
P = 128
B = 1024
S = 200
NTOK = B * S
QTOK = NTOK // 4

VB = 32768
VB4 = VB // 4
NFULL = VOCAB // VB
TAIL_V0 = NFULL * VB
TAIL = VOCAB - TAIL_V0
TAIL4 = TAIL // 4
NPREP = NFULL + 1


def _rne16(u):
    return (u + 0x7FFF + ((u >> 16) & 1)) >> 16


def _tc_prep(table_t):
    def body(t_ref, o_ref):
        ua = lax.bitcast_convert_type(t_ref[:H, :], jnp.uint32)
        ub = lax.bitcast_convert_type(t_ref[H:, :], jnp.uint32)
        word_t = (_rne16(ua) << 16) | _rne16(ub)
        word = jnp.transpose(word_t, (1, 0))
        wf32 = lax.bitcast_convert_type(word, jnp.float32)
        i = pl.program_id(0)

        @pl.when(i < NFULL)
        def _():
            for k in range(4):
                o_ref[:, k * H:(k + 1) * H] = wf32[k * VB4:(k + 1) * VB4, :]

        @pl.when(i == NFULL)
        def _():
            for k in range(4):
                o_ref[:TAIL4, k * H:(k + 1) * H] = (
                    wf32[k * TAIL4:(k + 1) * TAIL4, :])

    return pl.pallas_call(
        body,
        grid=(NPREP,),
        in_specs=[pl.BlockSpec((D, VB), lambda i: (0, i))],
        out_specs=pl.BlockSpec((VB4, P), lambda i: (i, 0)),
        out_shape=jax.ShapeDtypeStruct((VOCAB // 4, P), jnp.float32),
    )(table_t)


NC, NS = 2, 16
NW = NC * NS
TOK_PER_W = NTOK // NW
W_PER_Q = NW // 4
CHUNK = 1280
NCHUNK = TOK_PER_W // CHUNK


def _sc_gather(table_packed, idx_flat):
    mesh = plsc.VectorSubcoreMesh(core_axis_name="c", subcore_axis_name="s")

    @functools.partial(
        pl.kernel,
        mesh=mesh,
        out_type=jax.ShapeDtypeStruct((QTOK, P), jnp.float32),
        scratch_types=[
            pltpu.VMEM((CHUNK,), jnp.int32),
            pltpu.VMEM((CHUNK, H), jnp.float32),
            pltpu.SemaphoreType.DMA,
        ],
        compiler_params=pltpu.CompilerParams(use_tc_tiling_on_sc=False),
    )
    def k(table_hbm, idx_hbm, out_hbm, idx_v, rows_v, sem):
        wid = lax.axis_index("s") * NC + lax.axis_index("c")
        base = wid * TOK_PER_W
        row_base = (wid % W_PER_Q) * TOK_PER_W

        def body(i, carry):
            off = base + i * CHUNK
            row = row_base + i * CHUNK
            pltpu.sync_copy(idx_hbm.at[pl.ds(off, CHUNK)], idx_v)
            pltpu.async_copy(table_hbm.at[idx_v], rows_v, sem).wait()
            for q in range(4):
                @pl.when(wid // W_PER_Q == q)
                def _():
                    pltpu.sync_copy(
                        rows_v,
                        out_hbm.at[pl.ds(row, CHUNK), pl.ds(q * H, H)])
            return carry

        lax.fori_loop(0, NCHUNK, body, 0)

    return k(table_packed, idx_flat)


TB = 3200
NSTEPS = QTOK // TB


def _positional_encoding_np(seq_len, embed_dim):
    position = np.arange(0, seq_len, dtype=np.float32)[:, None]
    div_term = np.exp(
        np.arange(0, embed_dim, 2, dtype=np.float32)
        * (-math.log(10000.0) / embed_dim))
    pe = np.zeros((seq_len, embed_dim), dtype=np.float32)
    pe[:, 0::2] = np.sin(position * div_term)
    pe[:, 1::2] = np.cos(position * div_term)
    return pe


_PE_FULL = np.tile(_positional_encoding_np(S, D), (TB // S, 1))
_PE_A = np.ascontiguousarray(_PE_FULL[:, :H])
_PE_B = np.ascontiguousarray(_PE_FULL[:, H:])


def _tc_ln_proj(emb4, pe_a4, pe_b4, q128, wfa, wfb, bias):
    inv_d = 1.0 / D
    hi = np.uint32(0xFFFF0000)

    def body(e_ref, pa_ref, pb_ref, q_ref, wa_ref, wb_ref, b_ref, o_ref):
        w = lax.bitcast_convert_type(e_ref[...], jnp.uint32)
        a = lax.bitcast_convert_type(w & hi, jnp.float32) + pa_ref[...]
        bb = lax.bitcast_convert_type(w << 16, jnp.float32) + pb_ref[...]
        q = q_ref[...]
        mu = jnp.dot(a + bb, q, preferred_element_type=jnp.float32) * inv_d
        m2 = jnp.dot(a * a + bb * bb, q,
                     preferred_element_type=jnp.float32) * inv_d
        r = lax.rsqrt(m2 - mu * mu + 1e-5)
        na = (a - mu) * r
        nb = (bb - mu) * r
        wa = wa_ref[...]
        wb = wb_ref[...]
        bias_v = b_ref[...]
        for k in range(4):
            o_ref[k] = (
                jnp.dot(na[:, k * H:(k + 1) * H], wa,
                        preferred_element_type=jnp.float32)
                + jnp.dot(nb[:, k * H:(k + 1) * H], wb,
                          preferred_element_type=jnp.float32)
                + bias_v)

    return pl.pallas_call(
        body,
        grid=(NSTEPS,),
        in_specs=[
            pl.BlockSpec((TB, P), lambda i: (i, 0)),
            pl.BlockSpec((TB, P), lambda i: (0, 0)),
            pl.BlockSpec((TB, P), lambda i: (0, 0)),
            pl.BlockSpec((P, P), lambda i: (0, 0)),
            pl.BlockSpec((H, P), lambda i: (0, 0)),
            pl.BlockSpec((H, P), lambda i: (0, 0)),
            pl.BlockSpec((1, P), lambda i: (0, 0)),
        ],
        out_specs=pl.BlockSpec((4, TB, P), lambda i: (0, i, 0)),
        out_shape=jax.ShapeDtypeStruct((4, QTOK, P), jnp.float32),
    )(emb4, pe_a4, pe_b4, q128, wfa, wfb, bias)


def kernel(x, table, gamma, beta, W, b):
    packed = _tc_prep(jnp.transpose(table))
    table_rows = packed.reshape(VOCAB, H)
    v = x.reshape(NTOK).astype(jnp.int32)
    dv = v % VB
    main = (v - dv) + 4 * (dv % VB4) + dv // VB4
    dt = v - TAIL_V0
    tail = TAIL_V0 + 4 * (dt % TAIL4) + dt // TAIL4
    idx_flat = jnp.where(v < TAIL_V0, main, tail)
    emb4 = _sc_gather(table_rows, idx_flat)
    wf = gamma[:, None] * W
    wfa, wfb = wf[:H], wf[H:]
    bias = (beta @ W + b).reshape(1, P)
    pe_a4 = jnp.asarray(np.tile(_PE_A, (1, 4)))
    pe_b4 = jnp.asarray(np.tile(_PE_B, (1, 4)))
    q128 = jnp.asarray(np.kron(np.eye(4, dtype=np.float32),
                               np.ones((H, H), np.float32)))
    out = _tc_ln_proj(emb4, pe_a4, pe_b4, q128, wfa, wfb, bias)
    return out.reshape(B, S, P)
